# Initial kernel scaffold; baseline (speedup 1.0000x reference)
import functools

import jax
import jax.numpy as jnp
from jax import lax
from jax.experimental import pallas as pl
from jax.experimental.pallas import tpu as pltpu
from jax.experimental.pallas import tpu_sc as plsc

N_NODES = 10000
D_FEAT = 128
NUM_GRAPHS = 64
NC = 2
NS = 16
NW = NC * NS
CHUNK = 320
SUB = 80
CNT_W = 16


def _sc_body(x_hbm, b_hbm, out_sum, out_cnt,
             xb0, xb1, xb2, xb3, ix0, ix1, ix2, ix3,
             ones_buf, zrow, zcnt, shared_sum, shared_cnt):
    c = lax.axis_index("c")
    s = lax.axis_index("s")
    wid = s * NC + c
    base = wid * CHUNK

    xbs = [xb0, xb1, xb2, xb3]
    ixs = [ix0, ix1, ix2, ix3]

    zvec = jnp.zeros((16,), jnp.float32)
    onev = jnp.ones((16,), jnp.float32)
    for r in range(4):
        for j in range(D_FEAT // 16):
            zrow[r, pl.ds(j * 16, 16)] = zvec
        zcnt[r, :] = zvec

    def _fill_ones(i, carry):
        ones_buf[i, :] = onev
        return carry

    lax.fori_loop(0, SUB, _fill_ones, 0)

    pltpu.sync_copy(zrow, shared_sum.at[pl.ds(s * 4, 4)])
    pltpu.sync_copy(zcnt, shared_cnt.at[pl.ds(s * 4, 4)])
    plsc.subcore_barrier()

    def _do_chunk(k):
        off = base + k * SUB
        pltpu.sync_copy(x_hbm.at[pl.ds(off, SUB)], xbs[k])
        pltpu.sync_copy(b_hbm.at[pl.ds(off, SUB)], ixs[k])
        pltpu.sync_copy(xbs[k], shared_sum.at[ixs[k]], add=True)
        pltpu.sync_copy(ones_buf, shared_cnt.at[ixs[k]], add=True)

    _do_chunk(0)

    @pl.when(wid < NW - 1)
    def _():
        for k in range(1, CHUNK // SUB):
            _do_chunk(k)

    plsc.subcore_barrier()

    @pl.when(s == 0)
    def _():
        pltpu.sync_copy(shared_sum, out_sum.at[c])
        pltpu.sync_copy(shared_cnt, out_cnt.at[c])


_sc_segment_sum = functools.partial(
    pl.kernel,
    out_type=(
        jax.ShapeDtypeStruct((NC, NUM_GRAPHS, D_FEAT), jnp.float32),
        jax.ShapeDtypeStruct((NC, NUM_GRAPHS, CNT_W), jnp.float32),
    ),
    mesh=plsc.VectorSubcoreMesh(core_axis_name="c", subcore_axis_name="s"),
    scratch_types=(
        [pltpu.VMEM((SUB, D_FEAT), jnp.float32) for _ in range(4)]
        + [pltpu.VMEM((SUB,), jnp.int32) for _ in range(4)]
        + [
            pltpu.VMEM((SUB, CNT_W), jnp.float32),
            pltpu.VMEM((4, D_FEAT), jnp.float32),
            pltpu.VMEM((4, CNT_W), jnp.float32),
            pltpu.VMEM_SHARED((NUM_GRAPHS, D_FEAT), jnp.float32),
            pltpu.VMEM_SHARED((NUM_GRAPHS, CNT_W), jnp.float32),
        ]
    ),
)(_sc_body)


def _mlp_body(sum_ref, cnt_ref, u_ref, w1_ref, b1_ref, w2_ref, b2_ref, o_ref):
    sums = sum_ref[0] + sum_ref[1]
    counts = cnt_ref[0, :, 0:1] + cnt_ref[1, :, 0:1]
    mean = sums / jnp.maximum(counts, 1.0)
    u = u_ref[...]
    h = (
        jnp.dot(u, w1_ref[0:64, :], preferred_element_type=jnp.float32)
        + jnp.dot(mean, w1_ref[64:192, :], preferred_element_type=jnp.float32)
        + b1_ref[...]
    )
    h = jnp.maximum(h, 0.0)
    o_ref[...] = (
        jnp.dot(h, w2_ref[...], preferred_element_type=jnp.float32) + b2_ref[...]
    )


def _tc_mlp(sums_p, cnt_p, u, W1, b1, W2, b2):
    return pl.pallas_call(
        _mlp_body,
        out_shape=jax.ShapeDtypeStruct((u.shape[0], W2.shape[1]), jnp.float32),
    )(sums_p, cnt_p, u, W1, b1, W2, b2)


def kernel(x, edge_index, edge_attr, u, batch, W1, b1, W2, b2):
    sums_p, cnt_p = _sc_segment_sum(x, batch)
    return _tc_mlp(sums_p, cnt_p, u, W1, b1.reshape(1, -1), W2, b2.reshape(1, -1))


# speedup vs baseline: 4.3377x; 4.3377x over previous
import functools

import jax
import jax.numpy as jnp
from jax import lax
from jax.experimental import pallas as pl
from jax.experimental.pallas import tpu as pltpu
from jax.experimental.pallas import tpu_sc as plsc

N_NODES = 10000
D_FEAT = 128
NUM_GRAPHS = 64
NC = 2
NS = 16
NW = NC * NS
CHUNK = 320
SUB = 80
N_PAD = 10240


def _sc_body(x_hbm, b_hbm, out_sum,
             xb0, xb1, xb2, xb3, ix0, ix1, ix2, ix3,
             zrow, shared_sum):
    c = lax.axis_index("c")
    s = lax.axis_index("s")
    wid = s * NC + c
    base = wid * CHUNK

    xbs = [xb0, xb1, xb2, xb3]
    ixs = [ix0, ix1, ix2, ix3]

    zvec = jnp.zeros((16,), jnp.float32)
    for r in range(4):
        for j in range(D_FEAT // 16):
            zrow[r, pl.ds(j * 16, 16)] = zvec

    pltpu.sync_copy(zrow, shared_sum.at[pl.ds(s * 4, 4)])
    plsc.subcore_barrier()

    def _do_chunk(k):
        off = base + k * SUB
        pltpu.sync_copy(x_hbm.at[pl.ds(off, SUB)], xbs[k])
        pltpu.sync_copy(b_hbm.at[pl.ds(off, SUB)], ixs[k])
        pltpu.sync_copy(xbs[k], shared_sum.at[ixs[k]], add=True)

    _do_chunk(0)

    @pl.when(wid < NW - 1)
    def _():
        for k in range(1, CHUNK // SUB):
            _do_chunk(k)

    plsc.subcore_barrier()

    @pl.when(s == 0)
    def _():
        pltpu.sync_copy(shared_sum, out_sum.at[c])


_sc_segment_sum = functools.partial(
    pl.kernel,
    out_type=jax.ShapeDtypeStruct((NC, NUM_GRAPHS, D_FEAT), jnp.float32),
    mesh=plsc.VectorSubcoreMesh(core_axis_name="c", subcore_axis_name="s"),
    scratch_types=(
        [pltpu.VMEM((SUB, D_FEAT), jnp.float32) for _ in range(4)]
        + [pltpu.VMEM((SUB,), jnp.int32) for _ in range(4)]
        + [
            pltpu.VMEM((4, D_FEAT), jnp.float32),
            pltpu.VMEM_SHARED((NUM_GRAPHS, D_FEAT), jnp.float32),
        ]
    ),
)(_sc_body)


def _mlp_body(sum_ref, ids_ref, u_ref, w1_ref, b1_ref, w2_ref, b2_ref, o_ref):
    sums = sum_ref[0] + sum_ref[1]
    ids = ids_ref[...]
    gid = lax.broadcasted_iota(jnp.int32, (NUM_GRAPHS, 1), 0)
    onehot = (gid == ids).astype(jnp.float32)
    counts = jnp.sum(onehot, axis=1, keepdims=True)
    mean = sums / jnp.maximum(counts, 1.0)
    u = u_ref[...]
    h = (
        jnp.dot(u, w1_ref[0:64, :], preferred_element_type=jnp.float32)
        + jnp.dot(mean, w1_ref[64:192, :], preferred_element_type=jnp.float32)
        + b1_ref[...]
    )
    h = jnp.maximum(h, 0.0)
    o_ref[...] = (
        jnp.dot(h, w2_ref[...], preferred_element_type=jnp.float32) + b2_ref[...]
    )


def _tc_mlp(sums_p, ids_row, u, W1, b1, W2, b2):
    return pl.pallas_call(
        _mlp_body,
        out_shape=jax.ShapeDtypeStruct((u.shape[0], W2.shape[1]), jnp.float32),
    )(sums_p, ids_row, u, W1, b1, W2, b2)


def kernel(x, edge_index, edge_attr, u, batch, W1, b1, W2, b2):
    sums_p = _sc_segment_sum(x, batch)
    ids_row = jnp.concatenate(
        [batch, jnp.full((N_PAD - N_NODES,), NUM_GRAPHS, jnp.int32)]
    ).reshape(1, N_PAD)
    return _tc_mlp(
        sums_p, ids_row, u, W1, b1.reshape(1, -1), W2, b2.reshape(1, -1)
    )


# speedup vs baseline: 5.0209x; 1.1575x over previous
import functools

import jax
import jax.numpy as jnp
from jax import lax
from jax.experimental import pallas as pl
from jax.experimental.pallas import tpu as pltpu
from jax.experimental.pallas import tpu_sc as plsc

N_NODES = 10000
D_FEAT = 128
NUM_GRAPHS = 64
NC = 2
NS = 16
NW = NC * NS
CHUNK = 320
SUB = 80
N_PAD = 10240


def _sc_body(x_hbm, b_hbm, out_sum,
             xb0, xb1, xb2, xb3, ix0, ix1, ix2, ix3,
             zrow, shared_sum, sem_stage, sem_scat):
    c = lax.axis_index("c")
    s = lax.axis_index("s")
    wid = s * NC + c
    base = wid * CHUNK

    xbs = [xb0, xb1, xb2, xb3]
    ixs = [ix0, ix1, ix2, ix3]
    nk = CHUNK // SUB
    last = wid == NW - 1

    def _stage(k):
        off = base + k * SUB
        return (
            pltpu.async_copy(x_hbm.at[pl.ds(off, SUB)], xbs[k], sem_stage),
            pltpu.async_copy(b_hbm.at[pl.ds(off, SUB)], ixs[k], sem_stage),
        )

    stage0 = _stage(0)

    @pl.when(jnp.logical_not(last))
    def _():
        for k in range(1, nk):
            _stage(k)

    zvec = jnp.zeros((16,), jnp.float32)
    for r in range(4):
        for j in range(D_FEAT // 16):
            zrow[r, pl.ds(j * 16, 16)] = zvec

    pltpu.sync_copy(zrow, shared_sum.at[pl.ds(s * 4, 4)])
    plsc.subcore_barrier()

    stage0[0].wait()
    stage0[1].wait()
    d0 = pltpu.async_copy(xbs[0], shared_sum.at[ixs[0]], sem_scat, add=True)

    @pl.when(jnp.logical_not(last))
    def _():
        descs = []
        for k in range(1, nk):
            off = base + k * SUB
            pltpu.make_async_copy(x_hbm.at[pl.ds(off, SUB)], xbs[k],
                                  sem_stage).wait()
            pltpu.make_async_copy(b_hbm.at[pl.ds(off, SUB)], ixs[k],
                                  sem_stage).wait()
            descs.append(
                pltpu.async_copy(xbs[k], shared_sum.at[ixs[k]], sem_scat,
                                 add=True)
            )
        for d in descs:
            d.wait()

    d0.wait()
    plsc.subcore_barrier()

    @pl.when(s == 0)
    def _():
        pltpu.sync_copy(shared_sum, out_sum.at[c])


_sc_segment_sum = functools.partial(
    pl.kernel,
    out_type=jax.ShapeDtypeStruct((NC, NUM_GRAPHS, D_FEAT), jnp.float32),
    mesh=plsc.VectorSubcoreMesh(core_axis_name="c", subcore_axis_name="s"),
    scratch_types=(
        [pltpu.VMEM((SUB, D_FEAT), jnp.float32) for _ in range(4)]
        + [pltpu.VMEM((SUB,), jnp.int32) for _ in range(4)]
        + [
            pltpu.VMEM((4, D_FEAT), jnp.float32),
            pltpu.VMEM_SHARED((NUM_GRAPHS, D_FEAT), jnp.float32),
            pltpu.SemaphoreType.DMA,
            pltpu.SemaphoreType.DMA,
        ]
    ),
)(_sc_body)


def _mlp_body(sum_ref, ids_ref, u_ref, w1_ref, b1_ref, w2_ref, b2_ref, o_ref):
    sums = sum_ref[0] + sum_ref[1]
    ids = ids_ref[...]
    gid = lax.broadcasted_iota(jnp.int32, (NUM_GRAPHS, 1), 0)
    onehot = (gid == ids).astype(jnp.float32)
    counts = jnp.sum(onehot, axis=1, keepdims=True)
    mean = sums / jnp.maximum(counts, 1.0)
    u = u_ref[...]
    h = (
        jnp.dot(u, w1_ref[0:64, :], preferred_element_type=jnp.float32)
        + jnp.dot(mean, w1_ref[64:192, :], preferred_element_type=jnp.float32)
        + b1_ref[...]
    )
    h = jnp.maximum(h, 0.0)
    o_ref[...] = (
        jnp.dot(h, w2_ref[...], preferred_element_type=jnp.float32) + b2_ref[...]
    )


def _tc_mlp(sums_p, ids_row, u, W1, b1, W2, b2):
    return pl.pallas_call(
        _mlp_body,
        out_shape=jax.ShapeDtypeStruct((u.shape[0], W2.shape[1]), jnp.float32),
    )(sums_p, ids_row, u, W1, b1, W2, b2)


def kernel(x, edge_index, edge_attr, u, batch, W1, b1, W2, b2):
    sums_p = _sc_segment_sum(x, batch)
    ids_row = jnp.concatenate(
        [batch, jnp.full((N_PAD - N_NODES,), NUM_GRAPHS, jnp.int32)]
    ).reshape(1, N_PAD)
    return _tc_mlp(
        sums_p, ids_row, u, W1, b1.reshape(1, -1), W2, b2.reshape(1, -1)
    )
